# baseline (speedup 1.0000x reference)
import functools

import jax
import jax.numpy as jnp
from jax import lax
from jax.experimental import pallas as pl
from jax.experimental.pallas import tpu as pltpu
from jax.experimental.pallas import tpu_sc as plsc

_EPS = 1e-15
_B = 16384
_D = 64
_WIN = 20
_NC = 2
_NS = 16
_NW = _NC * _NS
_BPW = _B // _NW
_BLK = 128
_NBLK = _BPW // _BLK
_IPOS = _WIN
_INEG = _WIN + 1


def _sc_body(pos_hbm, neg_hbm, ctx_hbm, tw_hbm, cw_hbm, opos_hbm, oneg_hbm,
             cidx, idx_all, pos_rows, neg_rows, acc,
             stage_pos, stage_neg, sem_idx, sem_g, sem_a, sem_o):
    wid = lax.axis_index("s") * _NC + lax.axis_index("c")
    base = wid * _BPW

    cp = pltpu.async_copy(pos_hbm.at[pl.ds(base, _BPW)], idx_all.at[_IPOS],
                          sem_idx)
    cn = pltpu.async_copy(neg_hbm.at[pl.ds(base, _BPW)], idx_all.at[_INEG],
                          sem_idx)
    cp.wait()
    cn.wait()
    g1 = []
    for j in range(_NBLK):
        dst = pl.ds(j * _BLK, _BLK)
        g1.append(pltpu.async_copy(
            tw_hbm.at[idx_all.at[_IPOS, dst]], pos_rows.at[dst], sem_g))
        g1.append(pltpu.async_copy(
            tw_hbm.at[idx_all.at[_INEG, dst]], neg_rows.at[dst], sem_g))

    lanes = lax.iota(jnp.int32, 16)
    for h in range(_NBLK):
        ch = pltpu.async_copy(
            ctx_hbm.at[pl.ds(base + h * _BLK, _BLK), :], cidx, sem_idx)
        ch.wait()

        def xpose(g, carry):
            rows = g * 16 + lanes
            for w in range(_WIN):
                v = plsc.load_gather(cidx, [rows, jnp.full((16,), w, jnp.int32)])
                idx_all[w, pl.ds(h * _BLK + g * 16, 16)] = v
            return carry

        lax.fori_loop(0, _BLK // 16, xpose, 0)

    g2 = []
    for j in range(_NBLK):
        dst = pl.ds(j * _BLK, _BLK)
        g2.append(pltpu.async_copy(
            cw_hbm.at[idx_all.at[0, dst]], acc.at[dst], sem_a))
    for c in g2:
        c.wait()
    g3 = []
    for j in range(_NBLK):
        dst = pl.ds(j * _BLK, _BLK)
        for w in range(1, _WIN):
            g3.append(pltpu.async_copy(
                cw_hbm.at[idx_all.at[w, dst]], acc.at[dst], sem_a, add=True))
    for c in g1:
        c.wait()
    for c in g3:
        c.wait()

    def elem(e, carry):
        pv = None
        nv = None
        for k in range(_D // 16):
            sl = pl.ds(k * 16, 16)
            a = acc[e, sl]
            p = pos_rows[e, sl] * a
            n = neg_rows[e, sl] * a
            pv = p if pv is None else pv + p
            nv = n if nv is None else nv + n
        stage_pos[e] = pv
        stage_neg[e] = nv
        return carry

    lax.fori_loop(0, _BPW, elem, 0)

    o1 = pltpu.async_copy(stage_pos, opos_hbm.at[pl.ds(base, _BPW)], sem_o)
    o2 = pltpu.async_copy(stage_neg, oneg_hbm.at[pl.ds(base, _BPW)], sem_o)
    o1.wait()
    o2.wait()


_sc_cbow = functools.partial(
    pl.kernel,
    out_type=(jax.ShapeDtypeStruct((_B, 16), jnp.float32),
              jax.ShapeDtypeStruct((_B, 16), jnp.float32)),
    mesh=plsc.VectorSubcoreMesh(core_axis_name="c", subcore_axis_name="s",
                                num_cores=_NC, num_subcores=_NS),
    scratch_types=[
        pltpu.VMEM((_BLK, _WIN), jnp.int32),
        pltpu.VMEM((_WIN + 2, _BPW), jnp.int32),
        pltpu.VMEM((_BPW, _D), jnp.float32),
        pltpu.VMEM((_BPW, _D), jnp.float32),
        pltpu.VMEM((_BPW, _D), jnp.float32),
        pltpu.VMEM((_BPW, 16), jnp.float32),
        pltpu.VMEM((_BPW, 16), jnp.float32),
        pltpu.SemaphoreType.DMA,
        pltpu.SemaphoreType.DMA,
        pltpu.SemaphoreType.DMA,
        pltpu.SemaphoreType.DMA,
    ],
    compiler_params=pltpu.CompilerParams(use_tc_tiling_on_sc=False,
                                         needs_layout_passes=False),
)(_sc_body)


def _tc_finish(pp_ref, np_ref, out_ref):
    ps = jnp.sum(pp_ref[...], axis=1) * (1.0 / _WIN) + _EPS
    ns = jnp.sum(np_ref[...], axis=1) * (1.0 / _WIN) + _EPS
    pos_score = -jax.nn.log_sigmoid(ps)
    neg_score = -jax.nn.log_sigmoid(1.0 - ns)
    out_ref[0, 0] = jnp.mean(pos_score + neg_score)


def kernel(pos_nodes, neg_nodes, context_nodes, target_weight, context_weight):
    pos = pos_nodes.astype(jnp.int32)
    neg = neg_nodes.astype(jnp.int32)
    ctx = context_nodes.astype(jnp.int32)

    pp, nn = _sc_cbow(pos, neg, ctx, target_weight, context_weight)

    loss = pl.pallas_call(
        _tc_finish,
        out_shape=jax.ShapeDtypeStruct((1, 1), jnp.float32),
        out_specs=pl.BlockSpec(memory_space=pltpu.SMEM),
    )(pp, nn)
    return loss[0, 0]


# speedup vs baseline: 1.2044x; 1.2044x over previous
import functools

import jax
import jax.numpy as jnp
from jax import lax
from jax.experimental import pallas as pl
from jax.experimental.pallas import tpu as pltpu
from jax.experimental.pallas import tpu_sc as plsc

_EPS = 1e-15
_B = 16384
_D = 64
_WIN = 20
_NC = 2
_NS = 16
_NW = _NC * _NS
_BPW = _B // _NW
_BLK = 128
_HALF = 256
_IPOS = _WIN
_INEG = _WIN + 1


def _sc_body(pos_hbm, neg_hbm, ctxT_hbm, tab_hbm, opos_hbm, oneg_hbm,
             idx_all, pos_rows, neg_rows, acc, stage_p, stage_n,
             sem_idx, sem_g, sem_a, sem_o):
    wid = lax.axis_index("s") * _NC + lax.axis_index("c")
    base = wid * _BPW

    idx_cps = [
        pltpu.async_copy(pos_hbm.at[pl.ds(base, _BPW)], idx_all.at[_IPOS],
                         sem_idx),
        pltpu.async_copy(neg_hbm.at[pl.ds(base, _BPW)], idx_all.at[_INEG],
                         sem_idx),
    ]
    for w in range(_WIN):
        idx_cps.append(pltpu.async_copy(
            ctxT_hbm.at[w, pl.ds(base, _BPW)], idx_all.at[w], sem_idx))
    for c in idx_cps:
        c.wait()

    for half in range(_BPW // _HALF):
        hoff = half * _HALF
        g1 = []
        for j in range(_HALF // _BLK):
            src = pl.ds(hoff + j * _BLK, _BLK)
            dst = pl.ds(j * _BLK, _BLK)
            g1.append(pltpu.async_copy(
                tab_hbm.at[idx_all.at[_IPOS, src]], pos_rows.at[dst], sem_g))
            g1.append(pltpu.async_copy(
                tab_hbm.at[idx_all.at[_INEG, src]], neg_rows.at[dst], sem_g))
            g1.append(pltpu.async_copy(
                tab_hbm.at[idx_all.at[0, src]], acc.at[dst], sem_a))
        for c in g1:
            c.wait()
        g2 = []
        for j in range(_HALF // _BLK):
            src = pl.ds(hoff + j * _BLK, _BLK)
            dst = pl.ds(j * _BLK, _BLK)
            for w in range(1, _WIN):
                g2.append(pltpu.async_copy(
                    tab_hbm.at[idx_all.at[w, src]], acc.at[dst], sem_a,
                    add=True))
        for c in g2:
            c.wait()

        def elem(e, carry):
            pv = None
            nv = None
            for k in range(_D // 16):
                a = acc[e, pl.ds(_D + k * 16, 16)]
                p = pos_rows[e, pl.ds(k * 16, 16)] * a
                n = neg_rows[e, pl.ds(k * 16, 16)] * a
                pv = p if pv is None else pv + p
                nv = n if nv is None else nv + n
            row = (hoff + e) // 8
            lane = ((hoff + e) % 8) * 16
            stage_p[row, pl.ds(lane, 16)] = pv
            stage_n[row, pl.ds(lane, 16)] = nv
            return carry

        lax.fori_loop(0, _HALF, elem, 0)

    orow = wid * (_BPW // 8)
    o1 = pltpu.async_copy(stage_p, opos_hbm.at[pl.ds(orow, _BPW // 8)], sem_o)
    o2 = pltpu.async_copy(stage_n, oneg_hbm.at[pl.ds(orow, _BPW // 8)], sem_o)
    o1.wait()
    o2.wait()


_sc_cbow = functools.partial(
    pl.kernel,
    out_type=(jax.ShapeDtypeStruct((_B // 8, 128), jnp.float32),
              jax.ShapeDtypeStruct((_B // 8, 128), jnp.float32)),
    mesh=plsc.VectorSubcoreMesh(core_axis_name="c", subcore_axis_name="s",
                                num_cores=_NC, num_subcores=_NS),
    scratch_types=[
        pltpu.VMEM((_WIN + 2, _BPW), jnp.int32),
        pltpu.VMEM((_HALF, 2 * _D), jnp.float32),
        pltpu.VMEM((_HALF, 2 * _D), jnp.float32),
        pltpu.VMEM((_HALF, 2 * _D), jnp.float32),
        pltpu.VMEM((_BPW // 8, 128), jnp.float32),
        pltpu.VMEM((_BPW // 8, 128), jnp.float32),
        pltpu.SemaphoreType.DMA,
        pltpu.SemaphoreType.DMA,
        pltpu.SemaphoreType.DMA,
        pltpu.SemaphoreType.DMA,
    ],
    compiler_params=pltpu.CompilerParams(use_tc_tiling_on_sc=True),
)(_sc_body)


def _tc_finish(pp_ref, np_ref, out_ref):
    l = lax.broadcasted_iota(jnp.int32, (128, 8), 0)
    s = lax.broadcasted_iota(jnp.int32, (128, 8), 1)
    m = (l // 16 == s).astype(jnp.float32)
    ps = jnp.dot(pp_ref[...], m) * (1.0 / _WIN) + _EPS
    ns = jnp.dot(np_ref[...], m) * (1.0 / _WIN) + _EPS
    pos_score = -jax.nn.log_sigmoid(ps)
    neg_score = -jax.nn.log_sigmoid(1.0 - ns)
    out_ref[0, 0] = (jnp.sum(pos_score) + jnp.sum(neg_score)) * (1.0 / _B)


def kernel(pos_nodes, neg_nodes, context_nodes, target_weight, context_weight):
    pos = pos_nodes.astype(jnp.int32)
    neg = neg_nodes.astype(jnp.int32)
    ctxT = context_nodes.astype(jnp.int32).T
    tab = jnp.concatenate([target_weight, context_weight], axis=1)

    pp, nn = _sc_cbow(pos, neg, ctxT, tab)

    loss = pl.pallas_call(
        _tc_finish,
        out_shape=jax.ShapeDtypeStruct((1, 1), jnp.float32),
        out_specs=pl.BlockSpec(memory_space=pltpu.SMEM),
    )(pp, nn)
    return loss[0, 0]
